# baseline (speedup 1.0000x reference)
import functools

import jax
import jax.numpy as jnp
from jax import lax
from jax.experimental import pallas as pl
from jax.experimental.pallas import tpu as pltpu
from jax.experimental.pallas import tpu_sc as plsc

N_NODES = 50000
D = 64
HALF = 32
N_LAYERS = 3
NC, NS = 2, 16
LANES = 16
EB = 128
KB = 4
E = 800000

ACC_ROWS = 50016
RPT = ACC_ROWS // NS

KU = 2
UB = KU * EB
RB, IB, UNROLL = 3, 6, 6
_LCH = NS * UB * UNROLL
E_PAD = ((E + _LCH - 1) // _LCH) * _LCH
NB = E_PAD // (NS * UB)

_DCH = NC * NS * KB * EB
E_PAD_D = ((E + _DCH - 1) // _DCH) * _DCH
ROWS_TILE_D = E_PAD_D // (NC * NS * EB)
BLK_TILE_D = ROWS_TILE_D // KB

def _mesh():
    return plsc.VectorSubcoreMesh(core_axis_name="c", subcore_axis_name="s")


def _sc_params():
    return pltpu.CompilerParams(use_tc_tiling_on_sc=False)


def _deg_sc(row_hbm, ones_hbm, zero_hbm, deg_hbm, idx_v, ones_v, sem, acc):
    c = lax.axis_index("c")
    s = lax.axis_index("s")
    wid = s * NC + c
    pltpu.sync_copy(zero_hbm, acc.at[pl.ds(s * RPT, RPT)])
    pltpu.sync_copy(ones_hbm, ones_v)
    plsc.subcore_barrier()

    @pl.loop(0, BLK_TILE_D)
    def _(b):
        r0 = wid * ROWS_TILE_D + b * KB
        pltpu.sync_copy(row_hbm.at[pl.ds(r0, KB)], idx_v)
        for j in range(KB):
            pltpu.sync_copy(ones_v, acc.at[idx_v.at[j]], add=True)

    plsc.subcore_barrier()
    pltpu.sync_copy(acc.at[pl.ds(s * RPT, RPT)], deg_hbm.at[c, pl.ds(s * RPT, RPT)])


def _layer_sc(g_hbm, idx_hbm, zero_hbm, s_hbm, idx, rows, gsem, ssem, isem,
              acc):
    c = lax.axis_index("c")
    s = lax.axis_index("s")
    pltpu.sync_copy(zero_hbm, acc.at[pl.ds(s * RPT, RPT)])
    base = s * NB
    gtab = g_hbm.at[c]

    def pf(j, slot):
        pltpu.async_copy(idx_hbm.at[pl.ds((base + j) * 2 * KU, 2 * KU)],
                         idx.at[slot], isem)

    def wait_idx(j, slot):
        pltpu.make_async_copy(idx_hbm.at[pl.ds((base + j) * 2 * KU, 2 * KU)],
                              idx.at[slot], isem).wait()

    def issue_gathers(slot_i, slot_r):
        for k in range(KU):
            pltpu.async_copy(gtab.at[idx.at[slot_i, k]],
                             rows.at[slot_r].at[pl.ds(k * EB, EB)], gsem)

    def issue_scatters(slot_i, slot_r):
        for k in range(KU):
            pltpu.async_copy(rows.at[slot_r].at[pl.ds(k * EB, EB)],
                             acc.at[idx.at[slot_i, KU + k]], ssem, add=True)

    def wait_gathers(slot_r):
        pltpu.make_async_copy(gtab.at[pl.ds(0, UB)], rows.at[slot_r], gsem).wait()

    def wait_scatters(slot_r):
        pltpu.make_async_copy(rows.at[slot_r], acc.at[pl.ds(0, UB)], ssem).wait()

    plsc.subcore_barrier()
    pf(0, 0)
    pf(1, 1)
    pf(2, 2)

    @pl.loop(0, NB // UNROLL)
    def _(t):
        for u in range(UNROLL):
            j = t * UNROLL + u
            rs, isl = u % RB, u % IB

            @pl.when(j >= RB)
            def _():
                wait_scatters(rs)

            wait_idx(j, isl)
            issue_gathers(isl, rs)

            @pl.when(j + 3 < NB)
            def _():
                pf(j + 3, (u + 3) % IB)

            @pl.when(j >= 2)
            def _():
                wait_gathers((u - 2) % RB)
                issue_scatters((u - 2) % IB, (u - 2) % RB)

    for j in range(NB - 2, NB):
        wait_gathers(j % RB)
        issue_scatters(j % IB, j % RB)
    for j in range(NB - RB, NB):
        wait_scatters(j % RB)

    plsc.subcore_barrier()
    pltpu.sync_copy(acc.at[pl.ds(s * RPT, RPT)], s_hbm.at[c, pl.ds(s * RPT, RPT)])


@jax.jit
def _deg_call(row_d):
    ones = jnp.ones((EB, HALF), jnp.float32)
    zero = jnp.zeros((RPT, HALF), jnp.float32)
    return pl.kernel(
        _deg_sc,
        out_type=jax.ShapeDtypeStruct((NC, ACC_ROWS, HALF), jnp.float32),
        mesh=_mesh(),
        scratch_types=[
            pltpu.VMEM((KB, EB), jnp.int32),
            pltpu.VMEM((EB, HALF), jnp.float32),
            pltpu.SemaphoreType.DMA,
            pltpu.VMEM_SHARED((ACC_ROWS, HALF), jnp.float32),
        ],
        compiler_params=_sc_params(),
    )(row_d, ones, zero)


@jax.jit
def _layer_call(g, idx_hbm):
    zero = jnp.zeros((RPT, HALF), jnp.float32)
    return pl.kernel(
        _layer_sc,
        out_type=jax.ShapeDtypeStruct((NC, ACC_ROWS, HALF), jnp.float32),
        mesh=_mesh(),
        scratch_types=[
            pltpu.VMEM((IB, 2 * KU, EB), jnp.int32),
            pltpu.VMEM((RB, UB, HALF), jnp.float32),
            pltpu.SemaphoreType.DMA,
            pltpu.SemaphoreType.DMA,
            pltpu.SemaphoreType.DMA,
            pltpu.VMEM_SHARED((ACC_ROWS, HALF), jnp.float32),
        ],
        compiler_params=_sc_params(),
    )(g, idx_hbm, zero)


NF = N_NODES * HALF // 128
AF = ACC_ROWS * HALF // 128
_BF = 512
_GRIDF = ((NF + _BF - 1) // _BF,)
_spec_f1 = pl.BlockSpec((_BF, 128), lambda i: (i, 0))
_spec_f2 = pl.BlockSpec((NC, _BF, 128), lambda i: (0, i, 0))


def _prep_tc(deg_ref, x_ref, g_ref, dinv_ref):
    dsum = deg_ref[0] + deg_ref[1]
    dinv = jnp.where(dsum > 0, lax.rsqrt(jnp.where(dsum > 0, dsum, 1.0)), 0.0)
    dinv_ref[...] = dinv
    g_ref[0] = x_ref[0] * dinv
    g_ref[1] = x_ref[1] * dinv


def _scale_tc(s_ref, dinv_ref, g_ref):
    d2 = dinv_ref[...] * dinv_ref[...]
    g_ref[0] = s_ref[0] * d2
    g_ref[1] = s_ref[1] * d2


def _final_tc(x_ref, dinv_ref, s1_ref, s2_ref, s3_ref, o_ref):
    d = dinv_ref[...]
    o_ref[0] = 0.25 * (x_ref[0] + d * (s1_ref[0] + s2_ref[0] + s3_ref[0]))
    o_ref[1] = 0.25 * (x_ref[1] + d * (s1_ref[1] + s2_ref[1] + s3_ref[1]))


_F2 = jax.ShapeDtypeStruct((NC, NF, 128), jnp.float32)
_F1 = jax.ShapeDtypeStruct((NF, 128), jnp.float32)


@jax.jit
def _prep_call(deg_f, x_f):
    return pl.pallas_call(
        _prep_tc,
        grid=_GRIDF,
        in_specs=[_spec_f2, _spec_f2],
        out_specs=[_spec_f2, _spec_f1],
        out_shape=[_F2, _F1],
    )(deg_f, x_f)


@jax.jit
def _scale_call(s_f, dinv_f):
    return pl.pallas_call(
        _scale_tc,
        grid=_GRIDF,
        in_specs=[_spec_f2, _spec_f1],
        out_specs=_spec_f2,
        out_shape=_F2,
    )(s_f, dinv_f)


@jax.jit
def _final_call(x_f, dinv_f, s1, s2, s3):
    return pl.pallas_call(
        _final_tc,
        grid=_GRIDF,
        in_specs=[_spec_f2, _spec_f1, _spec_f2, _spec_f2, _spec_f2],
        out_specs=_spec_f2,
        out_shape=_F2,
    )(x_f, dinv_f, s1, s2, s3)


def _to_nodes(a_f):
    return a_f.reshape(NC, N_NODES, HALF)


def _flat(sacc):
    return sacc.reshape(NC, AF, 128)


def kernel(edge_index, emb_weight):
    row = edge_index[0].astype(jnp.int32)
    col = edge_index[1].astype(jnp.int32)
    row_g = jnp.concatenate(
        [row, jnp.zeros((E_PAD - E,), jnp.int32)]).reshape(-1, KU, EB)
    col_s = jnp.concatenate(
        [col, jnp.full((E_PAD - E,), N_NODES, jnp.int32)]).reshape(-1, KU, EB)
    idx_hbm = jnp.concatenate([row_g, col_s], axis=1).reshape(-1, EB)
    row_d = jnp.concatenate(
        [row, jnp.full((E_PAD_D - E,), N_NODES, jnp.int32)]).reshape(-1, EB)

    x_f = emb_weight.reshape(N_NODES, NC, HALF).transpose(1, 0, 2) \
                    .reshape(NC, NF, 128)

    deg = _deg_call(row_d)
    g_f, dinv_f = _prep_call(_flat(deg), x_f)
    s_flat = []
    for layer in range(N_LAYERS):
        sacc = _layer_call(_to_nodes(g_f), idx_hbm)
        s_flat.append(_flat(sacc))
        if layer < N_LAYERS - 1:
            g_f = _scale_call(s_flat[-1], dinv_f)
    out_f = _final_call(x_f, dinv_f, *s_flat)
    return out_f.reshape(NC, N_NODES, HALF).transpose(1, 0, 2) \
                .reshape(N_NODES, D)


# speedup vs baseline: 1.0003x; 1.0003x over previous
import functools

import jax
import jax.numpy as jnp
from jax import lax
from jax.experimental import pallas as pl
from jax.experimental.pallas import tpu as pltpu
from jax.experimental.pallas import tpu_sc as plsc

N_NODES = 50000
D = 64
HALF = 32
N_LAYERS = 3
NC, NS = 2, 16
LANES = 16
EB = 128
KB = 4
E = 800000

ACC_ROWS = 50016
RPT = ACC_ROWS // NS

KU = 2
UB = KU * EB
RB, IB, UNROLL = 3, 6, 6
_LCH = NS * UB * UNROLL
E_PAD = ((E + _LCH - 1) // _LCH) * _LCH
NB = E_PAD // (NS * UB)

_DCH = NC * NS * KB * EB
E_PAD_D = ((E + _DCH - 1) // _DCH) * _DCH
ROWS_TILE_D = E_PAD_D // (NC * NS * EB)
BLK_TILE_D = ROWS_TILE_D // KB

def _mesh():
    return plsc.VectorSubcoreMesh(core_axis_name="c", subcore_axis_name="s")


def _sc_params():
    return pltpu.CompilerParams(use_tc_tiling_on_sc=False)


def _deg_sc(row_hbm, ones_hbm, zero_hbm, deg_hbm, idx_v, ones_v, sem, acc):
    c = lax.axis_index("c")
    s = lax.axis_index("s")
    wid = s * NC + c
    pltpu.sync_copy(zero_hbm, acc.at[pl.ds(s * RPT, RPT)])
    pltpu.sync_copy(ones_hbm, ones_v)
    plsc.subcore_barrier()

    @pl.loop(0, BLK_TILE_D)
    def _(b):
        r0 = wid * ROWS_TILE_D + b * KB
        pltpu.sync_copy(row_hbm.at[pl.ds(r0, KB)], idx_v)
        for j in range(KB):
            pltpu.sync_copy(ones_v, acc.at[idx_v.at[j]], add=True)

    plsc.subcore_barrier()
    pltpu.sync_copy(acc.at[pl.ds(s * RPT, RPT)], deg_hbm.at[c, pl.ds(s * RPT, RPT)])


def _layer_sc(g_hbm, idx_hbm, zero_hbm, s_hbm, idx, rows, gsem, ssem, isem,
              acc):
    c = lax.axis_index("c")
    s = lax.axis_index("s")
    pltpu.sync_copy(zero_hbm, acc.at[pl.ds(s * RPT, RPT)])
    base = s * NB
    gtab = g_hbm.at[c]

    def pf(j, slot):
        pltpu.async_copy(idx_hbm.at[pl.ds((base + j) * 2 * KU, 2 * KU)],
                         idx.at[slot], isem)

    def wait_idx(j, slot):
        pltpu.make_async_copy(idx_hbm.at[pl.ds((base + j) * 2 * KU, 2 * KU)],
                              idx.at[slot], isem).wait()

    def issue_gathers(slot_i, slot_r):
        for k in range(KU):
            pltpu.async_copy(gtab.at[idx.at[slot_i, k]],
                             rows.at[slot_r].at[pl.ds(k * EB, EB)], gsem)

    def issue_scatters(slot_i, slot_r):
        for k in range(KU):
            pltpu.async_copy(rows.at[slot_r].at[pl.ds(k * EB, EB)],
                             acc.at[idx.at[slot_i, KU + k]], ssem, add=True)

    def wait_gathers(slot_r):
        pltpu.make_async_copy(gtab.at[pl.ds(0, UB)], rows.at[slot_r], gsem).wait()

    def wait_scatters(slot_r):
        pltpu.make_async_copy(rows.at[slot_r], acc.at[pl.ds(0, UB)], ssem).wait()

    plsc.subcore_barrier()
    pf(0, 0)
    pf(1, 1)

    @pl.loop(0, NB // UNROLL)
    def _(t):
        for u in range(UNROLL):
            j = t * UNROLL + u
            rs, isl = u % RB, u % IB

            @pl.when(j >= RB)
            def _():
                wait_scatters(rs)

            wait_idx(j, isl)
            issue_gathers(isl, rs)

            @pl.when(j + 2 < NB)
            def _():
                pf(j + 2, (u + 2) % IB)

            @pl.when(j >= 2)
            def _():
                wait_gathers((u - 2) % RB)
                issue_scatters((u - 2) % IB, (u - 2) % RB)

    for j in range(NB - 2, NB):
        wait_gathers(j % RB)
        issue_scatters(j % IB, j % RB)
    for j in range(NB - RB, NB):
        wait_scatters(j % RB)

    plsc.subcore_barrier()
    pltpu.sync_copy(acc.at[pl.ds(s * RPT, RPT)], s_hbm.at[c, pl.ds(s * RPT, RPT)])


@jax.jit
def _deg_call(row_d):
    ones = jnp.ones((EB, HALF), jnp.float32)
    zero = jnp.zeros((RPT, HALF), jnp.float32)
    return pl.kernel(
        _deg_sc,
        out_type=jax.ShapeDtypeStruct((NC, ACC_ROWS, HALF), jnp.float32),
        mesh=_mesh(),
        scratch_types=[
            pltpu.VMEM((KB, EB), jnp.int32),
            pltpu.VMEM((EB, HALF), jnp.float32),
            pltpu.SemaphoreType.DMA,
            pltpu.VMEM_SHARED((ACC_ROWS, HALF), jnp.float32),
        ],
        compiler_params=_sc_params(),
    )(row_d, ones, zero)


@jax.jit
def _layer_call(g, idx_hbm):
    zero = jnp.zeros((RPT, HALF), jnp.float32)
    return pl.kernel(
        _layer_sc,
        out_type=jax.ShapeDtypeStruct((NC, ACC_ROWS, HALF), jnp.float32),
        mesh=_mesh(),
        scratch_types=[
            pltpu.VMEM((IB, 2 * KU, EB), jnp.int32),
            pltpu.VMEM((RB, UB, HALF), jnp.float32),
            pltpu.SemaphoreType.DMA,
            pltpu.SemaphoreType.DMA,
            pltpu.SemaphoreType.DMA,
            pltpu.VMEM_SHARED((ACC_ROWS, HALF), jnp.float32),
        ],
        compiler_params=_sc_params(),
    )(g, idx_hbm, zero)


NF = N_NODES * HALF // 128
AF = ACC_ROWS * HALF // 128
_BF = 512
_GRIDF = ((NF + _BF - 1) // _BF,)
_spec_f1 = pl.BlockSpec((_BF, 128), lambda i: (i, 0))
_spec_f2 = pl.BlockSpec((NC, _BF, 128), lambda i: (0, i, 0))


def _prep_tc(deg_ref, x_ref, g_ref, dinv_ref):
    dsum = deg_ref[0] + deg_ref[1]
    dinv = jnp.where(dsum > 0, lax.rsqrt(jnp.where(dsum > 0, dsum, 1.0)), 0.0)
    dinv_ref[...] = dinv
    g_ref[0] = x_ref[0] * dinv
    g_ref[1] = x_ref[1] * dinv


def _scale_tc(s_ref, dinv_ref, g_ref):
    d2 = dinv_ref[...] * dinv_ref[...]
    g_ref[0] = s_ref[0] * d2
    g_ref[1] = s_ref[1] * d2


def _final_tc(x_ref, dinv_ref, s1_ref, s2_ref, s3_ref, o_ref):
    d = dinv_ref[...]
    o_ref[0] = 0.25 * (x_ref[0] + d * (s1_ref[0] + s2_ref[0] + s3_ref[0]))
    o_ref[1] = 0.25 * (x_ref[1] + d * (s1_ref[1] + s2_ref[1] + s3_ref[1]))


_F2 = jax.ShapeDtypeStruct((NC, NF, 128), jnp.float32)
_F1 = jax.ShapeDtypeStruct((NF, 128), jnp.float32)


@jax.jit
def _prep_call(deg_f, x_f):
    return pl.pallas_call(
        _prep_tc,
        grid=_GRIDF,
        in_specs=[_spec_f2, _spec_f2],
        out_specs=[_spec_f2, _spec_f1],
        out_shape=[_F2, _F1],
    )(deg_f, x_f)


@jax.jit
def _scale_call(s_f, dinv_f):
    return pl.pallas_call(
        _scale_tc,
        grid=_GRIDF,
        in_specs=[_spec_f2, _spec_f1],
        out_specs=_spec_f2,
        out_shape=_F2,
    )(s_f, dinv_f)


@jax.jit
def _final_call(x_f, dinv_f, s1, s2, s3):
    return pl.pallas_call(
        _final_tc,
        grid=_GRIDF,
        in_specs=[_spec_f2, _spec_f1, _spec_f2, _spec_f2, _spec_f2],
        out_specs=_spec_f2,
        out_shape=_F2,
    )(x_f, dinv_f, s1, s2, s3)


def _to_nodes(a_f):
    return a_f.reshape(NC, N_NODES, HALF)


def _flat(sacc):
    return sacc.reshape(NC, AF, 128)


def kernel(edge_index, emb_weight):
    row = edge_index[0].astype(jnp.int32)
    col = edge_index[1].astype(jnp.int32)
    row_g = jnp.concatenate(
        [row, jnp.zeros((E_PAD - E,), jnp.int32)]).reshape(-1, KU, EB)
    col_s = jnp.concatenate(
        [col, jnp.full((E_PAD - E,), N_NODES, jnp.int32)]).reshape(-1, KU, EB)
    idx_hbm = jnp.concatenate([row_g, col_s], axis=1).reshape(-1, EB)
    row_d = jnp.concatenate(
        [row, jnp.full((E_PAD_D - E,), N_NODES, jnp.int32)]).reshape(-1, EB)

    x_f = emb_weight.reshape(N_NODES, NC, HALF).transpose(1, 0, 2) \
                    .reshape(NC, NF, 128)

    deg = _deg_call(row_d)
    g_f, dinv_f = _prep_call(_flat(deg), x_f)
    s_flat = []
    for layer in range(N_LAYERS):
        sacc = _layer_call(_to_nodes(g_f), idx_hbm)
        s_flat.append(_flat(sacc))
        if layer < N_LAYERS - 1:
            g_f = _scale_call(s_flat[-1], dinv_f)
    out_f = _final_call(x_f, dinv_f, *s_flat)
    return out_f.reshape(NC, N_NODES, HALF).transpose(1, 0, 2) \
                .reshape(N_NODES, D)
